# Initial kernel scaffold; baseline (speedup 1.0000x reference)
import jax
import jax.numpy as jnp
from jax.experimental import pallas as pl


def kernel(x, edge_index, W1, b1, W2, b2):
    raise NotImplementedError("write your pallas kernel here")


# speedup vs baseline: 35.0416x; 35.0416x over previous
import functools

import jax
import jax.numpy as jnp
from jax import lax
from jax.experimental import pallas as pl
from jax.experimental.pallas import tpu as pltpu
import jax.experimental.pallas.tpu_sc as plsc

N = 10000
E = 320000
DIN = 128
DH = 16
DOUT = 128

NC = 2
NS = 16
NW = NC * NS
L = 16

CHUNK = 128
CHUNKS = -(-E // (NW * CHUNK))
E_PAD = NW * CHUNKS * CHUNK
N_PAD = 10112
STRIPE = N_PAD // NS

_SC_MESH = plsc.VectorSubcoreMesh(core_axis_name="c", subcore_axis_name="s")
_SC_PARAMS = pltpu.CompilerParams(use_tc_tiling_on_sc=False)


def _zero_stripe(bounce_v, accum_sh, sid):
    zvec = jnp.zeros((L,), jnp.float32)

    def zb(i, c):
        bounce_v[i, :] = zvec
        return c

    lax.fori_loop(0, STRIPE, zb, 0)
    pltpu.sync_copy(bounce_v, accum_sh.at[pl.ds(sid * STRIPE, STRIPE)])


def _readout_stripe(accum_sh, out_hbm, cid, sid):
    pltpu.sync_copy(
        accum_sh.at[pl.ds(sid * STRIPE, STRIPE)],
        out_hbm.at[cid, pl.ds(sid * STRIPE, STRIPE)],
    )


def _deg_body(dst_hbm, out_hbm, dst_v, rows_v, bounce_v, accum_sh):
    cid = lax.axis_index("c")
    sid = lax.axis_index("s")
    wid = sid * NC + cid
    pltpu.sync_copy(dst_hbm.at[wid], dst_v)

    e1 = jnp.where(lax.iota(jnp.int32, L) == 0, 1.0, 0.0).astype(jnp.float32)

    def fill(i, c):
        rows_v[i, :] = e1
        return c

    lax.fori_loop(0, CHUNK, fill, 0)
    _zero_stripe(bounce_v, accum_sh, sid)
    plsc.subcore_barrier()

    def chunk(j, c):
        pltpu.sync_copy(rows_v, accum_sh.at[dst_v.at[j]], add=True)
        return c

    lax.fori_loop(0, CHUNKS, chunk, 0)
    plsc.subcore_barrier()
    _readout_stripe(accum_sh, out_hbm, cid, sid)


_sc_degree = pl.kernel(
    _deg_body,
    out_type=jax.ShapeDtypeStruct((NC, N_PAD, L), jnp.float32),
    mesh=_SC_MESH,
    compiler_params=_SC_PARAMS,
    scratch_types=[
        pltpu.VMEM((CHUNKS, CHUNK), jnp.int32),
        pltpu.VMEM((CHUNK, L), jnp.float32),
        pltpu.VMEM((STRIPE, L), jnp.float32),
        pltpu.VMEM_SHARED((N_PAD, L), jnp.float32),
    ],
)


def _agg_body(src_hbm, dst_hbm, g_hbm, out_hbm, src_v, dst_v, rows_v, bounce_v,
              accum_sh, sem):
    cid = lax.axis_index("c")
    sid = lax.axis_index("s")
    wid = sid * NC + cid
    pltpu.sync_copy(src_hbm.at[wid], src_v)
    pltpu.sync_copy(dst_hbm.at[wid], dst_v)
    _zero_stripe(bounce_v, accum_sh, sid)
    plsc.subcore_barrier()

    def chunk(j, c):
        pltpu.async_copy(g_hbm.at[src_v.at[j]], rows_v, sem).wait()
        pltpu.sync_copy(rows_v, accum_sh.at[dst_v.at[j]], add=True)
        return c

    lax.fori_loop(0, CHUNKS, chunk, 0)
    plsc.subcore_barrier()
    _readout_stripe(accum_sh, out_hbm, cid, sid)


_sc_agg = pl.kernel(
    _agg_body,
    out_type=jax.ShapeDtypeStruct((NC, N_PAD, L), jnp.float32),
    mesh=_SC_MESH,
    compiler_params=_SC_PARAMS,
    scratch_types=[
        pltpu.VMEM((CHUNKS, CHUNK), jnp.int32),
        pltpu.VMEM((CHUNKS, CHUNK), jnp.int32),
        pltpu.VMEM((CHUNK, L), jnp.float32),
        pltpu.VMEM((STRIPE, L), jnp.float32),
        pltpu.VMEM_SHARED((N_PAD, L), jnp.float32),
        pltpu.SemaphoreType.DMA,
    ],
)


def _tc_pre_body(x_ref, w1_ref, degp_ref, g1_ref, dinv_ref):
    h1 = jnp.dot(x_ref[...], w1_ref[...], preferred_element_type=jnp.float32)
    deg = jnp.sum(degp_ref[0] + degp_ref[1], axis=1, keepdims=True) + 1.0
    dinvb = jnp.broadcast_to(lax.rsqrt(deg), (N_PAD, DH))
    dinv_ref[...] = dinvb
    g1_ref[...] = h1 * dinvb


_tc_pre = pl.pallas_call(
    _tc_pre_body,
    out_shape=[
        jax.ShapeDtypeStruct((N_PAD, DH), jnp.float32),
        jax.ShapeDtypeStruct((N_PAD, DH), jnp.float32),
    ],
)


def _tc_mid_body(a1p_ref, g1_ref, dinv_ref, b1_ref, g2_ref):
    s = (a1p_ref[0] + a1p_ref[1] + g1_ref[...]) * dinv_ref[...]
    z1 = jnp.maximum(s + b1_ref[...], 0.0)
    g2_ref[...] = z1 * dinv_ref[...]


_tc_mid = pl.pallas_call(
    _tc_mid_body,
    out_shape=jax.ShapeDtypeStruct((N_PAD, DH), jnp.float32),
)


def _tc_post_body(a2p_ref, g2_ref, dinv_ref, w2_ref, b2_ref, out_ref):
    a2 = (a2p_ref[0] + a2p_ref[1] + g2_ref[...]) * dinv_ref[...]
    h = jnp.dot(a2, w2_ref[...], preferred_element_type=jnp.float32)
    out_ref[...] = jnp.maximum(h + b2_ref[...], 0.0)


_tc_post = pl.pallas_call(
    _tc_post_body,
    out_shape=jax.ShapeDtypeStruct((N_PAD, DOUT), jnp.float32),
)


def kernel(x, edge_index, W1, b1, W2, b2):
    src = edge_index[0].astype(jnp.int32)
    dst = edge_index[1].astype(jnp.int32)
    pad_e = E_PAD - E
    src_t = jnp.concatenate([src, jnp.zeros((pad_e,), jnp.int32)])
    src_t = src_t.reshape(NW, CHUNKS, CHUNK)
    dst_t = jnp.concatenate([dst, jnp.full((pad_e,), N, jnp.int32)])
    dst_t = dst_t.reshape(NW, CHUNKS, CHUNK)
    x_pad = jnp.concatenate([x, jnp.zeros((N_PAD - N, DIN), jnp.float32)])

    degp = _sc_degree(dst_t)
    g1, dinvb = _tc_pre(x_pad, W1, degp)
    a1p = _sc_agg(src_t, dst_t, g1)
    g2 = _tc_mid(a1p, g1, dinvb, b1.reshape(1, DH))
    a2p = _sc_agg(src_t, dst_t, g2)
    out = _tc_post(a2p, g2, dinvb, W2, b2.reshape(1, DOUT))
    return out[:N]


# speedup vs baseline: 36.4969x; 1.0415x over previous
import functools

import jax
import jax.numpy as jnp
from jax import lax
from jax.experimental import pallas as pl
from jax.experimental.pallas import tpu as pltpu
import jax.experimental.pallas.tpu_sc as plsc

N = 10000
E = 320000
DIN = 128
DH = 16
DOUT = 128

NC = 2
NS = 16
NW = NC * NS
L = 16

CHUNK = 128
CHUNKS = -(-E // (NW * CHUNK))
E_PAD = NW * CHUNKS * CHUNK
CHUNKS_ALLOC = CHUNKS + 1
N_PAD = 10112
STRIPE = N_PAD // NS

_SC_MESH = plsc.VectorSubcoreMesh(core_axis_name="c", subcore_axis_name="s")
_SC_PARAMS = pltpu.CompilerParams(use_tc_tiling_on_sc=False)


def _zero_stripe(bounce_v, accum_sh, sid):
    zvec = jnp.zeros((L,), jnp.float32)

    def zb(i, c):
        bounce_v[i, :] = zvec
        return c

    lax.fori_loop(0, STRIPE, zb, 0)
    pltpu.sync_copy(bounce_v, accum_sh.at[pl.ds(sid * STRIPE, STRIPE)])


def _readout_stripe(accum_sh, out_hbm, cid, sid):
    pltpu.sync_copy(
        accum_sh.at[pl.ds(sid * STRIPE, STRIPE)],
        out_hbm.at[cid, pl.ds(sid * STRIPE, STRIPE)],
    )


def _deg_body(dst_hbm, out_hbm, dst_v, rows_v, bounce_v, accum_sh):
    cid = lax.axis_index("c")
    sid = lax.axis_index("s")
    wid = sid * NC + cid
    pltpu.sync_copy(dst_hbm.at[wid], dst_v)

    e1 = jnp.where(lax.iota(jnp.int32, L) == 0, 1.0, 0.0).astype(jnp.float32)

    def fill(i, c):
        rows_v[i, :] = e1
        return c

    lax.fori_loop(0, CHUNK, fill, 0)
    _zero_stripe(bounce_v, accum_sh, sid)
    plsc.subcore_barrier()

    def chunk(j, c):
        pltpu.sync_copy(rows_v, accum_sh.at[dst_v.at[j]], add=True)
        return c

    lax.fori_loop(0, CHUNKS, chunk, 0)
    plsc.subcore_barrier()
    _readout_stripe(accum_sh, out_hbm, cid, sid)


_sc_degree = pl.kernel(
    _deg_body,
    out_type=jax.ShapeDtypeStruct((NC, N_PAD, L), jnp.float32),
    mesh=_SC_MESH,
    compiler_params=_SC_PARAMS,
    scratch_types=[
        pltpu.VMEM((CHUNKS_ALLOC, CHUNK), jnp.int32),
        pltpu.VMEM((CHUNK, L), jnp.float32),
        pltpu.VMEM((STRIPE, L), jnp.float32),
        pltpu.VMEM_SHARED((N_PAD, L), jnp.float32),
    ],
)


def _agg_body(src_hbm, dst_hbm, g_hbm, out_hbm, src_v, dst_v, rows_v, bounce_v,
              accum_sh, sems):
    cid = lax.axis_index("c")
    sid = lax.axis_index("s")
    wid = sid * NC + cid
    pltpu.sync_copy(src_hbm.at[wid], src_v)
    pltpu.sync_copy(dst_hbm.at[wid], dst_v)
    _zero_stripe(bounce_v, accum_sh, sid)
    plsc.subcore_barrier()

    pltpu.async_copy(g_hbm.at[src_v.at[0]], rows_v.at[0], sems.at[0])

    def chunk(j, c):
        b = lax.rem(j, 2)
        nb = lax.rem(j + 1, 2)
        pltpu.async_copy(g_hbm.at[src_v.at[j + 1]], rows_v.at[nb], sems.at[nb])
        pltpu.make_async_copy(g_hbm.at[src_v.at[j]], rows_v.at[b],
                              sems.at[b]).wait()
        pltpu.sync_copy(rows_v.at[b], accum_sh.at[dst_v.at[j]], add=True)
        return c

    lax.fori_loop(0, CHUNKS, chunk, 0)
    b = CHUNKS % 2
    pltpu.make_async_copy(g_hbm.at[src_v.at[CHUNKS]], rows_v.at[b],
                          sems.at[b]).wait()
    plsc.subcore_barrier()
    _readout_stripe(accum_sh, out_hbm, cid, sid)


_sc_agg = pl.kernel(
    _agg_body,
    out_type=jax.ShapeDtypeStruct((NC, N_PAD, L), jnp.float32),
    mesh=_SC_MESH,
    compiler_params=_SC_PARAMS,
    scratch_types=[
        pltpu.VMEM((CHUNKS_ALLOC, CHUNK), jnp.int32),
        pltpu.VMEM((CHUNKS_ALLOC, CHUNK), jnp.int32),
        pltpu.VMEM((2, CHUNK, L), jnp.float32),
        pltpu.VMEM((STRIPE, L), jnp.float32),
        pltpu.VMEM_SHARED((N_PAD, L), jnp.float32),
        pltpu.SemaphoreType.DMA((2,)),
    ],
)


def _tc_pre_body(x_ref, w1_ref, degp_ref, g1_ref, dinv_ref):
    h1 = jnp.dot(x_ref[...], w1_ref[...], preferred_element_type=jnp.float32)
    deg = jnp.sum(degp_ref[0] + degp_ref[1], axis=1, keepdims=True) + 1.0
    dinvb = jnp.broadcast_to(lax.rsqrt(deg), (N_PAD, DH))
    dinv_ref[...] = dinvb
    g1_ref[...] = h1 * dinvb


_tc_pre = pl.pallas_call(
    _tc_pre_body,
    out_shape=[
        jax.ShapeDtypeStruct((N_PAD, DH), jnp.float32),
        jax.ShapeDtypeStruct((N_PAD, DH), jnp.float32),
    ],
)


def _tc_mid_body(a1p_ref, g1_ref, dinv_ref, b1_ref, g2_ref):
    s = (a1p_ref[0] + a1p_ref[1] + g1_ref[...]) * dinv_ref[...]
    z1 = jnp.maximum(s + b1_ref[...], 0.0)
    g2_ref[...] = z1 * dinv_ref[...]


_tc_mid = pl.pallas_call(
    _tc_mid_body,
    out_shape=jax.ShapeDtypeStruct((N_PAD, DH), jnp.float32),
)


def _tc_post_body(a2p_ref, g2_ref, dinv_ref, w2_ref, b2_ref, out_ref):
    a2 = (a2p_ref[0] + a2p_ref[1] + g2_ref[...]) * dinv_ref[...]
    h = jnp.dot(a2, w2_ref[...], preferred_element_type=jnp.float32)
    out_ref[...] = jnp.maximum(h + b2_ref[...], 0.0)


_tc_post = pl.pallas_call(
    _tc_post_body,
    out_shape=jax.ShapeDtypeStruct((N_PAD, DOUT), jnp.float32),
)


def kernel(x, edge_index, W1, b1, W2, b2):
    src = edge_index[0].astype(jnp.int32)
    dst = edge_index[1].astype(jnp.int32)
    pad_e = E_PAD - E
    src_t = jnp.concatenate([src, jnp.zeros((pad_e,), jnp.int32)])
    src_t = src_t.reshape(NW, CHUNKS, CHUNK)
    src_t = jnp.concatenate(
        [src_t, jnp.zeros((NW, 1, CHUNK), jnp.int32)], axis=1)
    dst_t = jnp.concatenate([dst, jnp.full((pad_e,), N, jnp.int32)])
    dst_t = dst_t.reshape(NW, CHUNKS, CHUNK)
    dst_t = jnp.concatenate(
        [dst_t, jnp.full((NW, 1, CHUNK), N, jnp.int32)], axis=1)
    x_pad = jnp.concatenate([x, jnp.zeros((N_PAD - N, DIN), jnp.float32)])

    degp = _sc_degree(dst_t)
    g1, dinvb = _tc_pre(x_pad, W1, degp)
    a1p = _sc_agg(src_t, dst_t, g1)
    g2 = _tc_mid(a1p, g1, dinvb, b1.reshape(1, DH))
    a2p = _sc_agg(src_t, dst_t, g2)
    out = _tc_post(a2p, g2, dinvb, W2, b2.reshape(1, DOUT))
    return out[:N]


# speedup vs baseline: 58.8221x; 1.6117x over previous
import functools

import jax
import jax.numpy as jnp
from jax import lax
from jax.experimental import pallas as pl
from jax.experimental.pallas import tpu as pltpu
import jax.experimental.pallas.tpu_sc as plsc

N = 10000
E = 320000
DIN = 128
DH = 16
DOUT = 128

NC = 2
NS = 16
NW = NC * NS
L = 16

CHUNK = 128
CHUNKS = -(-E // (NW * CHUNK))
E_PAD = NW * CHUNKS * CHUNK
CHUNKS_ALLOC = CHUNKS + 1
N_PAD = 10112
STRIPE = N_PAD // NS

_SC_MESH = plsc.VectorSubcoreMesh(core_axis_name="c", subcore_axis_name="s")
_SC_PARAMS = pltpu.CompilerParams(use_tc_tiling_on_sc=False)


def _zero_stripe(bounce_v, accum_sh, sid):
    zvec = jnp.zeros((L,), jnp.float32)

    def zb(i, c):
        bounce_v[i, :] = zvec
        return c

    lax.fori_loop(0, STRIPE, zb, 0)
    pltpu.sync_copy(bounce_v, accum_sh.at[pl.ds(sid * STRIPE, STRIPE)])


def _readout_stripe(accum_sh, out_hbm, cid, sid):
    pltpu.sync_copy(
        accum_sh.at[pl.ds(sid * STRIPE, STRIPE)],
        out_hbm.at[cid, pl.ds(sid * STRIPE, STRIPE)],
    )


def _deg_body(dst_hbm, out_hbm, dst_v, rows_v, bounce_v, accum_sh):
    cid = lax.axis_index("c")
    sid = lax.axis_index("s")
    wid = sid * NC + cid
    pltpu.sync_copy(dst_hbm.at[wid], dst_v)

    e1 = jnp.where(lax.iota(jnp.int32, L) == 0, 1.0, 0.0).astype(jnp.float32)

    def fill(i, c):
        rows_v[i, :] = e1
        return c

    lax.fori_loop(0, CHUNK, fill, 0)
    _zero_stripe(bounce_v, accum_sh, sid)
    plsc.subcore_barrier()

    def chunk(j, c):
        pltpu.sync_copy(rows_v, accum_sh.at[dst_v.at[j]], add=True)
        return c

    lax.fori_loop(0, CHUNKS, chunk, 0)
    plsc.subcore_barrier()
    _readout_stripe(accum_sh, out_hbm, cid, sid)


_sc_degree = pl.kernel(
    _deg_body,
    out_type=jax.ShapeDtypeStruct((NC, N_PAD, L), jnp.float32),
    mesh=_SC_MESH,
    compiler_params=_SC_PARAMS,
    scratch_types=[
        pltpu.VMEM((CHUNKS_ALLOC, CHUNK), jnp.int32),
        pltpu.VMEM((CHUNK, L), jnp.float32),
        pltpu.VMEM((STRIPE, L), jnp.float32),
        pltpu.VMEM_SHARED((N_PAD, L), jnp.float32),
    ],
)


def _agg_body(src_hbm, dst_hbm, g_hbm, out_hbm, src_v, dst_v, rows_v, bounce_v,
              g_sh, accum_sh, sems):
    cid = lax.axis_index("c")
    sid = lax.axis_index("s")
    wid = sid * NC + cid
    pltpu.sync_copy(src_hbm.at[wid], src_v)
    pltpu.sync_copy(dst_hbm.at[wid], dst_v)
    pltpu.sync_copy(g_hbm.at[pl.ds(sid * STRIPE, STRIPE)],
                    g_sh.at[pl.ds(sid * STRIPE, STRIPE)])
    _zero_stripe(bounce_v, accum_sh, sid)
    plsc.subcore_barrier()

    pltpu.async_copy(g_sh.at[src_v.at[0]], rows_v.at[0], sems.at[0])

    def chunk(j, c):
        b = lax.rem(j, 2)
        nb = lax.rem(j + 1, 2)
        pltpu.async_copy(g_sh.at[src_v.at[j + 1]], rows_v.at[nb], sems.at[nb])
        pltpu.make_async_copy(g_sh.at[src_v.at[j]], rows_v.at[b],
                              sems.at[b]).wait()
        pltpu.sync_copy(rows_v.at[b], accum_sh.at[dst_v.at[j]], add=True)
        return c

    lax.fori_loop(0, CHUNKS, chunk, 0)
    b = CHUNKS % 2
    pltpu.make_async_copy(g_sh.at[src_v.at[CHUNKS]], rows_v.at[b],
                          sems.at[b]).wait()
    plsc.subcore_barrier()
    _readout_stripe(accum_sh, out_hbm, cid, sid)


_sc_agg = pl.kernel(
    _agg_body,
    out_type=jax.ShapeDtypeStruct((NC, N_PAD, L), jnp.float32),
    mesh=_SC_MESH,
    compiler_params=_SC_PARAMS,
    scratch_types=[
        pltpu.VMEM((CHUNKS_ALLOC, CHUNK), jnp.int32),
        pltpu.VMEM((CHUNKS_ALLOC, CHUNK), jnp.int32),
        pltpu.VMEM((2, CHUNK, L), jnp.float32),
        pltpu.VMEM((STRIPE, L), jnp.float32),
        pltpu.VMEM_SHARED((N_PAD, L), jnp.float32),
        pltpu.VMEM_SHARED((N_PAD, L), jnp.float32),
        pltpu.SemaphoreType.DMA((2,)),
    ],
)


def _tc_pre_body(x_ref, w1_ref, degp_ref, g1_ref, dinv_ref):
    h1 = jnp.dot(x_ref[...], w1_ref[...], preferred_element_type=jnp.float32)
    deg = jnp.sum(degp_ref[0] + degp_ref[1], axis=1, keepdims=True) + 1.0
    dinvb = jnp.broadcast_to(lax.rsqrt(deg), (N_PAD, DH))
    dinv_ref[...] = dinvb
    g1_ref[...] = h1 * dinvb


_tc_pre = pl.pallas_call(
    _tc_pre_body,
    out_shape=[
        jax.ShapeDtypeStruct((N_PAD, DH), jnp.float32),
        jax.ShapeDtypeStruct((N_PAD, DH), jnp.float32),
    ],
)


def _tc_mid_body(a1p_ref, g1_ref, dinv_ref, b1_ref, g2_ref):
    s = (a1p_ref[0] + a1p_ref[1] + g1_ref[...]) * dinv_ref[...]
    z1 = jnp.maximum(s + b1_ref[...], 0.0)
    g2_ref[...] = z1 * dinv_ref[...]


_tc_mid = pl.pallas_call(
    _tc_mid_body,
    out_shape=jax.ShapeDtypeStruct((N_PAD, DH), jnp.float32),
)


def _tc_post_body(a2p_ref, g2_ref, dinv_ref, w2_ref, b2_ref, out_ref):
    a2 = (a2p_ref[0] + a2p_ref[1] + g2_ref[...]) * dinv_ref[...]
    h = jnp.dot(a2, w2_ref[...], preferred_element_type=jnp.float32)
    out_ref[...] = jnp.maximum(h + b2_ref[...], 0.0)


_tc_post = pl.pallas_call(
    _tc_post_body,
    out_shape=jax.ShapeDtypeStruct((N_PAD, DOUT), jnp.float32),
)


def kernel(x, edge_index, W1, b1, W2, b2):
    src = edge_index[0].astype(jnp.int32)
    dst = edge_index[1].astype(jnp.int32)
    pad_e = E_PAD - E
    src_t = jnp.concatenate([src, jnp.zeros((pad_e,), jnp.int32)])
    src_t = src_t.reshape(NW, CHUNKS, CHUNK)
    src_t = jnp.concatenate(
        [src_t, jnp.zeros((NW, 1, CHUNK), jnp.int32)], axis=1)
    dst_t = jnp.concatenate([dst, jnp.full((pad_e,), N, jnp.int32)])
    dst_t = dst_t.reshape(NW, CHUNKS, CHUNK)
    dst_t = jnp.concatenate(
        [dst_t, jnp.full((NW, 1, CHUNK), N, jnp.int32)], axis=1)
    x_pad = jnp.concatenate([x, jnp.zeros((N_PAD - N, DIN), jnp.float32)])

    degp = _sc_degree(dst_t)
    g1, dinvb = _tc_pre(x_pad, W1, degp)
    a1p = _sc_agg(src_t, dst_t, g1)
    g2 = _tc_mid(a1p, g1, dinvb, b1.reshape(1, DH))
    a2p = _sc_agg(src_t, dst_t, g2)
    out = _tc_post(a2p, g2, dinvb, W2, b2.reshape(1, DOUT))
    return out[:N]
